# baseline (speedup 1.0000x reference)
import functools

import jax
import jax.numpy as jnp
from jax import lax
from jax.experimental import pallas as pl
from jax.experimental.pallas import tpu as pltpu
from jax.experimental.pallas import tpu_sc as plsc

NUM_CORES = 2
NUM_SUBCORES = 16
NUM_WORKERS = NUM_CORES * NUM_SUBCORES
BATCH = 16384
EMBED_DIM = 64
B_PER_W = BATCH // NUM_WORKERS
CHUNK = 256
NCHUNK = B_PER_W // CHUNK
LANES = 16
TROWS = 500000
TCOLS = 2 * EMBED_DIM

_mesh = plsc.VectorSubcoreMesh(core_axis_name="c", subcore_axis_name="s")


@functools.partial(
    pl.kernel,
    mesh=_mesh,
    compiler_params=pltpu.CompilerParams(use_tc_tiling_on_sc=True),
    out_type=[
        jax.ShapeDtypeStruct((BATCH,), jnp.float32),
        jax.ShapeDtypeStruct((BATCH,), jnp.float32),
    ],
    scratch_types=[
        pltpu.VMEM((B_PER_W,), jnp.int32),
        pltpu.VMEM((B_PER_W,), jnp.int32),
        pltpu.VMEM((B_PER_W,), jnp.int32),
        pltpu.VMEM((CHUNK,), jnp.int32),
        pltpu.VMEM((CHUNK,), jnp.int32),
        pltpu.VMEM((CHUNK,), jnp.int32),
        pltpu.VMEM((CHUNK, TCOLS), jnp.float32),
        pltpu.VMEM((CHUNK, TCOLS), jnp.float32),
        pltpu.VMEM((CHUNK, TCOLS), jnp.float32),
        pltpu.VMEM((B_PER_W,), jnp.float32),
        pltpu.VMEM((B_PER_W,), jnp.float32),
        pltpu.SemaphoreType.DMA,
    ],
)
def _sc_fused(u_tab, i_tab, uid, pid, nid, pos_out, neg_out,
              uidx, pidx, nidx, g_u, g_p, g_n,
              urows, prows, nrows, pos_v, neg_v, sem):
    wid = lax.axis_index("s") * NUM_CORES + lax.axis_index("c")
    base = wid * B_PER_W
    sl = pl.ds(base, B_PER_W)
    pltpu.sync_copy(uid.at[sl], uidx)
    pltpu.sync_copy(pid.at[sl], pidx)
    pltpu.sync_copy(nid.at[sl], nidx)

    lane = lax.iota(jnp.int32, LANES)
    one = jnp.full((LANES,), 1, jnp.int32)
    dnums = lax.GatherDimensionNumbers(
        offset_dims=(), collapsed_slice_dims=(0,), start_index_map=(0,))

    def lane_sum(v):
        for k in (8, 4, 2, 1):
            v = v + lax.gather(v, (lane ^ k)[:, None], dnums, (1,),
                               mode=lax.GatherScatterMode.PROMISE_IN_BOUNDS)
        return v

    def bcast(v, j):
        return lax.gather(v, jnp.full((LANES, 1), j, jnp.int32), dnums, (1,),
                          mode=lax.GatherScatterMode.PROMISE_IN_BOUNDS)

    for ch in range(NCHUNK):
        cbase = ch * CHUNK

        @pl.loop(0, CHUNK // LANES)
        def _(t):
            s16 = pl.ds(t * LANES, LANES)
            sg16 = pl.ds(cbase + t * LANES, LANES)
            g_u[s16] = lax.shift_right_logical(uidx[sg16], 1)
            g_p[s16] = lax.shift_right_logical(pidx[sg16], 1)
            g_n[s16] = lax.shift_right_logical(nidx[sg16], 1)

        cu = pltpu.async_copy(u_tab.at[g_u], urows, sem)
        cp = pltpu.async_copy(i_tab.at[g_p], prows, sem)
        cn = pltpu.async_copy(i_tab.at[g_n], nrows, sem)
        cu.wait()
        cp.wait()
        cn.wait()

        @pl.loop(0, CHUNK // LANES)
        def _(g):
            sg16 = pl.ds(cbase + g * LANES, LANES)
            upar = (uidx[sg16] & one).astype(jnp.float32)
            ppar = (pidx[sg16] & one).astype(jnp.float32)
            npar = (nidx[sg16] & one).astype(jnp.float32)
            accp = jnp.zeros((LANES,), jnp.float32)
            accn = jnp.zeros((LANES,), jnp.float32)
            for j in range(LANES):
                li = g * LANES + j
                mu = bcast(upar, j)
                mp = bcast(ppar, j)
                mn = bcast(npar, j)
                sp = jnp.zeros((LANES,), jnp.float32)
                sn = jnp.zeros((LANES,), jnp.float32)
                for c in range(EMBED_DIM // LANES):
                    ulo = urows[li, pl.ds(c * LANES, LANES)]
                    uhi = urows[li, pl.ds(EMBED_DIM + c * LANES, LANES)]
                    u = ulo + (uhi - ulo) * mu
                    plo = prows[li, pl.ds(c * LANES, LANES)]
                    phi = prows[li, pl.ds(EMBED_DIM + c * LANES, LANES)]
                    p = plo + (phi - plo) * mp
                    nlo = nrows[li, pl.ds(c * LANES, LANES)]
                    nhi = nrows[li, pl.ds(EMBED_DIM + c * LANES, LANES)]
                    n = nlo + (nhi - nlo) * mn
                    sp = sp + u * p
                    sn = sn + u * n
                mask = lane == j
                accp = jnp.where(mask, lane_sum(sp), accp)
                accn = jnp.where(mask, lane_sum(sn), accn)
            pos_v[sg16] = accp
            neg_v[sg16] = accn

    wp = pltpu.async_copy(pos_v, pos_out.at[sl], sem)
    wn = pltpu.async_copy(neg_v, neg_out.at[sl], sem)
    wp.wait()
    wn.wait()


def kernel(user_ids, pos_item_ids, neg_item_ids, user_table, item_table):
    uid = user_ids.astype(jnp.int32)
    pid = pos_item_ids.astype(jnp.int32)
    nid = neg_item_ids.astype(jnp.int32)
    ut2 = user_table.reshape(TROWS, TCOLS)
    it2 = item_table.reshape(TROWS, TCOLS)
    return tuple(_sc_fused(ut2, it2, uid, pid, nid))


# speedup vs baseline: 1.0675x; 1.0675x over previous
import functools

import jax
import jax.numpy as jnp
from jax import lax
from jax.experimental import pallas as pl
from jax.experimental.pallas import tpu as pltpu
from jax.experimental.pallas import tpu_sc as plsc

NUM_CORES = 2
NUM_SUBCORES = 16
NUM_WORKERS = NUM_CORES * NUM_SUBCORES
BATCH = 16384
EMBED_DIM = 64
B_PER_W = BATCH // NUM_WORKERS
CHUNK = 256
NCHUNK = B_PER_W // CHUNK
LANES = 16
TROWS = 1000000
TCOLS = 128

_mesh = plsc.VectorSubcoreMesh(core_axis_name="c", subcore_axis_name="s")


@functools.partial(
    pl.kernel,
    mesh=_mesh,
    compiler_params=pltpu.CompilerParams(use_tc_tiling_on_sc=True),
    out_type=[
        jax.ShapeDtypeStruct((BATCH,), jnp.float32),
        jax.ShapeDtypeStruct((BATCH,), jnp.float32),
    ],
    scratch_types=[
        pltpu.VMEM((B_PER_W,), jnp.int32),
        pltpu.VMEM((B_PER_W,), jnp.int32),
        pltpu.VMEM((B_PER_W,), jnp.int32),
        pltpu.VMEM((CHUNK, TCOLS), jnp.float32),
        pltpu.VMEM((CHUNK, TCOLS), jnp.float32),
        pltpu.VMEM((CHUNK, TCOLS), jnp.float32),
        pltpu.VMEM((B_PER_W,), jnp.float32),
        pltpu.VMEM((B_PER_W,), jnp.float32),
        pltpu.SemaphoreType.DMA,
    ],
)
def _sc_fused(u_tab, i_tab, uid, pid, nid, pos_out, neg_out,
              uidx, pidx, nidx, urows, prows, nrows, pos_v, neg_v, sem):
    wid = lax.axis_index("s") * NUM_CORES + lax.axis_index("c")
    base = wid * B_PER_W
    sl = pl.ds(base, B_PER_W)
    pltpu.sync_copy(uid.at[sl], uidx)
    pltpu.sync_copy(pid.at[sl], pidx)
    pltpu.sync_copy(nid.at[sl], nidx)

    lane = lax.iota(jnp.int32, LANES)
    dnums = lax.GatherDimensionNumbers(
        offset_dims=(), collapsed_slice_dims=(0,), start_index_map=(0,))

    def lane_sum(v):
        for k in (8, 4, 2, 1):
            v = v + lax.gather(v, (lane ^ k)[:, None], dnums, (1,),
                               mode=lax.GatherScatterMode.PROMISE_IN_BOUNDS)
        return v

    for ch in range(NCHUNK):
        cbase = ch * CHUNK
        cu = pltpu.async_copy(u_tab.at[uidx.at[pl.ds(cbase, CHUNK)]], urows, sem)
        cp = pltpu.async_copy(i_tab.at[pidx.at[pl.ds(cbase, CHUNK)]], prows, sem)
        cn = pltpu.async_copy(i_tab.at[nidx.at[pl.ds(cbase, CHUNK)]], nrows, sem)
        cu.wait()
        cp.wait()
        cn.wait()

        @pl.loop(0, CHUNK // LANES)
        def _(g):
            sg16 = pl.ds(cbase + g * LANES, LANES)
            accp = jnp.zeros((LANES,), jnp.float32)
            accn = jnp.zeros((LANES,), jnp.float32)
            for j in range(LANES):
                li = g * LANES + j
                sp = jnp.zeros((LANES,), jnp.float32)
                sn = jnp.zeros((LANES,), jnp.float32)
                for c in range(EMBED_DIM // LANES):
                    u = urows[li, pl.ds(c * LANES, LANES)]
                    sp = sp + u * prows[li, pl.ds(c * LANES, LANES)]
                    sn = sn + u * nrows[li, pl.ds(c * LANES, LANES)]
                mask = lane == j
                accp = jnp.where(mask, lane_sum(sp), accp)
                accn = jnp.where(mask, lane_sum(sn), accn)
            pos_v[sg16] = accp
            neg_v[sg16] = accn

    wp = pltpu.async_copy(pos_v, pos_out.at[sl], sem)
    wn = pltpu.async_copy(neg_v, neg_out.at[sl], sem)
    wp.wait()
    wn.wait()


def kernel(user_ids, pos_item_ids, neg_item_ids, user_table, item_table):
    uid = user_ids.astype(jnp.int32)
    pid = pos_item_ids.astype(jnp.int32)
    nid = neg_item_ids.astype(jnp.int32)
    ut2 = jnp.pad(user_table, ((0, 0), (0, TCOLS - EMBED_DIM)))
    it2 = jnp.pad(item_table, ((0, 0), (0, TCOLS - EMBED_DIM)))
    return tuple(_sc_fused(ut2, it2, uid, pid, nid))
